# baseline (speedup 1.0000x reference)
import functools

import jax
import jax.numpy as jnp
from jax import lax
from jax.experimental import pallas as pl
from jax.experimental.pallas import tpu as pltpu
from jax.experimental.pallas import tpu_sc as plsc

VOCAB = 1_000_000
HIDDEN = 64
BATCH = 4096
HIST = 200

_NW = 32
_ROWS_PER_W = BATCH // _NW
_HP = 128
_TB = 2048
_HT = HIDDEN // 8


def _transpose_pad(emb_t):
  def body(in_ref, out_ref):
    x = in_ref[...]
    eye = jnp.eye(HIDDEN, dtype=jnp.float32)
    xt = lax.dot_general(x, eye, (((0,), (0,)), ((), ())),
                         precision=lax.Precision.HIGHEST,
                         preferred_element_type=jnp.float32)
    out_ref[:, 0:HIDDEN] = xt
    out_ref[:, HIDDEN:_HP] = jnp.zeros((_TB, _HP - HIDDEN), jnp.float32)

  return pl.pallas_call(
      body,
      grid=(pl.cdiv(VOCAB, _TB),),
      in_specs=[pl.BlockSpec((HIDDEN, _TB), lambda i: (0, i))],
      out_specs=pl.BlockSpec((_TB, _HP), lambda i: (i, 0)),
      out_shape=jax.ShapeDtypeStruct((VOCAB, _HP), jnp.float32),
  )(emb_t)


def _make_gather():
  mesh = plsc.VectorSubcoreMesh(core_axis_name="c", subcore_axis_name="s")

  @functools.partial(
      pl.kernel,
      out_type=jax.ShapeDtypeStruct((HIST, _HT, _NW, 8, 128), jnp.float32),
      mesh=mesh,
      scratch_types=[
          pltpu.VMEM((_ROWS_PER_W, HIST), jnp.int32),
          pltpu.VMEM((HIST * _ROWS_PER_W,), jnp.int32),
          pltpu.VMEM((_ROWS_PER_W, _HP), jnp.float32),
          pltpu.VMEM((_ROWS_PER_W, _HP), jnp.float32),
          pltpu.VMEM((_HT, 8, 128), jnp.float32),
          pltpu.VMEM((_HT, 8, 128), jnp.float32),
          pltpu.SemaphoreType.DMA,
          pltpu.SemaphoreType.DMA,
          pltpu.SemaphoreType.DMA,
          pltpu.SemaphoreType.DMA,
      ],
      compiler_params=pltpu.CompilerParams(
          use_tc_tiling_on_sc=True, disable_bounds_checks=True,
          needs_layout_passes=False),
  )
  def gather_kernel(emb_hbm, idx_hbm, out_hbm, idx2d, idx_t,
                    rows0, rows1, tb0, tb1,
                    gsem0, gsem1, ssem0, ssem1):
    wid = lax.axis_index("s") * 2 + lax.axis_index("c")
    base = wid * _ROWS_PER_W

    pltpu.sync_copy(idx_hbm.at[pl.ds(base, _ROWS_PER_W), :], idx2d)

    bbs = [lax.iota(jnp.int32, 16) + 16 * c for c in range(8)]

    def trans_idx(t, _):
      for c in range(8):
        v = plsc.load_gather(idx2d, [bbs[c], jnp.full((16,), t, jnp.int32)])
        idx_t[pl.ds(t * _ROWS_PER_W + 16 * c, 16)] = v
      return 0

    lax.fori_loop(0, HIST, trans_idx, 0, unroll=False)

    rows = (rows0, rows1)
    tb = (tb0, tb1)
    gsem = (gsem0, gsem1)
    ssem = (ssem0, ssem1)

    def gather(t, b):
      pltpu.make_async_copy(
          emb_hbm.at[idx_t.at[pl.ds(t * _ROWS_PER_W, _ROWS_PER_W)]],
          rows[b], gsem[b]).start()

    def gather_wait(b):
      pltpu.make_async_copy(
          emb_hbm.at[idx_t.at[pl.ds(0, _ROWS_PER_W)]], rows[b],
          gsem[b]).wait()

    def transpose(b):
      for a in range(_HT):
        for r in range(8):
          h = jnp.full((16,), 8 * a + r, jnp.int32)
          for c in range(8):
            tb[b][a, r, pl.ds(16 * c, 16)] = (
                plsc.load_gather(rows[b], [bbs[c], h]))

    def store(t, b):
      pltpu.make_async_copy(
          tb[b], out_hbm.at[t, :, wid], ssem[b]).start()

    def store_wait(b):
      pltpu.make_async_copy(
          tb[b], out_hbm.at[0, :, wid], ssem[b]).wait()

    gather(0, 0)
    gather(1, 1)

    def body(t, b):
      gather_wait(b)

      @pl.when(t >= 2)
      def _():
        store_wait(b)

      transpose(b)
      store(t, b)

      @pl.when(t + 2 < HIST)
      def _():
        gather(t + 2, b)

    def pair(k, _):
      g = 2 * k
      for b in range(2):
        body(g + b, b)
      return 0

    lax.fori_loop(0, HIST // 2, pair, 0, unroll=False)

    store_wait(0)
    store_wait(1)

  return gather_kernel


_gather = _make_gather()


def kernel(input_ids, emb):
  emb_pad = _transpose_pad(emb.T)
  out5 = _gather(emb_pad, input_ids.astype(jnp.int32))
  return out5.transpose(2, 4, 0, 1, 3).reshape(BATCH, HIST, HIDDEN)


# speedup vs baseline: 1.1752x; 1.1752x over previous
import functools

import jax
import jax.numpy as jnp
from jax import lax
from jax.experimental import pallas as pl
from jax.experimental.pallas import tpu as pltpu
from jax.experimental.pallas import tpu_sc as plsc

VOCAB = 1_000_000
HIDDEN = 64
BATCH = 4096
HIST = 200

_NW = 32
_ROWS_PER_W = BATCH // _NW
_HP = 128
_TB = 2048
_HT = HIDDEN // 8


def _transpose_pad(emb_t):
  def body(in_ref, out_ref):
    x = in_ref[...]
    eye = jnp.eye(HIDDEN, dtype=jnp.bfloat16)
    hi = x.astype(jnp.bfloat16)
    lo = (x - hi.astype(jnp.float32)).astype(jnp.bfloat16)
    dims = (((0,), (0,)), ((), ()))
    xt = (lax.dot_general(hi, eye, dims, preferred_element_type=jnp.float32)
          + lax.dot_general(lo, eye, dims,
                            preferred_element_type=jnp.float32))
    out_ref[:, 0:HIDDEN] = xt

  return pl.pallas_call(
      body,
      grid=(pl.cdiv(VOCAB, _TB),),
      in_specs=[pl.BlockSpec((HIDDEN, _TB), lambda i: (0, i))],
      out_specs=pl.BlockSpec((_TB, _HP), lambda i: (i, 0)),
      out_shape=jax.ShapeDtypeStruct((VOCAB, _HP), jnp.float32),
  )(emb_t)


def _make_gather():
  mesh = plsc.VectorSubcoreMesh(core_axis_name="c", subcore_axis_name="s")

  @functools.partial(
      pl.kernel,
      out_type=jax.ShapeDtypeStruct((HIST, _HT, _NW, 8, 128), jnp.float32),
      mesh=mesh,
      scratch_types=[
          pltpu.VMEM((_ROWS_PER_W, HIST), jnp.int32),
          pltpu.VMEM((HIST * _ROWS_PER_W,), jnp.int32),
          pltpu.VMEM((_ROWS_PER_W, _HP), jnp.float32),
          pltpu.VMEM((_ROWS_PER_W, _HP), jnp.float32),
          pltpu.VMEM((_HT, 8, 128), jnp.float32),
          pltpu.VMEM((_HT, 8, 128), jnp.float32),
          pltpu.VMEM((_HT * 8 * 8 * 16,), jnp.int32),
          pltpu.SemaphoreType.DMA,
          pltpu.SemaphoreType.DMA,
          pltpu.SemaphoreType.DMA,
          pltpu.SemaphoreType.DMA,
      ],
      compiler_params=pltpu.CompilerParams(
          use_tc_tiling_on_sc=True, disable_bounds_checks=True,
          needs_layout_passes=False),
  )
  def gather_kernel(emb_hbm, idx_hbm, out_hbm, idx2d, idx_t,
                    rows0, rows1, tb0, tb1, offtab,
                    gsem0, gsem1, ssem0, ssem1):
    wid = lax.axis_index("s") * 2 + lax.axis_index("c")
    base = wid * _ROWS_PER_W

    pltpu.sync_copy(idx_hbm.at[pl.ds(base, _ROWS_PER_W), :], idx2d)

    bbs = [lax.iota(jnp.int32, 16) + 16 * c for c in range(8)]

    def trans_idx(t, _):
      for c in range(8):
        v = plsc.load_gather(idx2d, [bbs[c], jnp.full((16,), t, jnp.int32)])
        idx_t[pl.ds(t * _ROWS_PER_W + 16 * c, 16)] = v
      return 0

    lax.fori_loop(0, HIST, trans_idx, 0, unroll=False)

    rows = (rows0, rows1)
    tb = (tb0, tb1)
    gsem = (gsem0, gsem1)
    ssem = (ssem0, ssem1)

    def gather(t, b):
      pltpu.make_async_copy(
          emb_hbm.at[idx_t.at[pl.ds(t * _ROWS_PER_W, _ROWS_PER_W)]],
          rows[b], gsem[b]).start()

    def gather_wait(b):
      pltpu.make_async_copy(
          emb_hbm.at[idx_t.at[pl.ds(0, _ROWS_PER_W)]], rows[b],
          gsem[b]).wait()

    zeros16 = jnp.zeros((16,), jnp.int32)
    bump = lax.iota(jnp.int32, 16) * 128

    def fill_offs(h, _):
      for c in range(8):
        offtab[pl.ds((h * 8 + c) * 16, 16)] = bump + (2048 * c + h)
      return 0

    lax.fori_loop(0, HIDDEN, fill_offs, 0, unroll=False)

    def transpose(b):
      for a in range(_HT):
        for r in range(8):
          for c in range(8):
            k = ((8 * a + r) * 8 + c) * 16
            tb[b][a, r, pl.ds(16 * c, 16)] = (
                plsc.load_gather(rows[b],
                                 [zeros16, offtab[pl.ds(k, 16)]]))

    def store(t, b):
      pltpu.make_async_copy(
          tb[b], out_hbm.at[t, :, wid], ssem[b]).start()

    def store_wait(b):
      pltpu.make_async_copy(
          tb[b], out_hbm.at[0, :, wid], ssem[b]).wait()

    gather(0, 0)
    gather(1, 1)

    def body(t, b):
      gather_wait(b)

      @pl.when(t >= 2)
      def _():
        store_wait(b)

      transpose(b)
      store(t, b)

      @pl.when(t + 2 < HIST)
      def _():
        gather(t + 2, b)

    def pair(k, _):
      g = 2 * k
      for b in range(2):
        body(g + b, b)
      return 0

    lax.fori_loop(0, HIST // 2, pair, 0, unroll=False)

    store_wait(0)
    store_wait(1)

  return gather_kernel


_gather = _make_gather()


def kernel(input_ids, emb):
  emb_pad = _transpose_pad(emb.T)
  out5 = _gather(emb_pad, input_ids.astype(jnp.int32))
  return out5.transpose(2, 4, 0, 1, 3).reshape(BATCH, HIST, HIDDEN)


# speedup vs baseline: 1.9076x; 1.6232x over previous
import functools

import jax
import jax.numpy as jnp
from jax import lax
from jax.experimental import pallas as pl
from jax.experimental.pallas import tpu as pltpu
from jax.experimental.pallas import tpu_sc as plsc

VOCAB = 1_000_000
HIDDEN = 64
BATCH = 4096
HIST = 200

_NW = 32
_ROWS_PER_W = BATCH // _NW
_HP = 128
_CHUNK = 160
_B_PER_W = _ROWS_PER_W * HIST
_NCHUNK = _B_PER_W // _CHUNK
_TB = 2048
_ISTAGE = 32
_NCH = 13
_CHS = tuple(list(range(0, 192, 16)) + [184])


def _transpose_pad(emb_t):
  def body(in_ref, out_ref):
    x = in_ref[...]
    eye = jnp.eye(HIDDEN, dtype=jnp.bfloat16)
    hi = x.astype(jnp.bfloat16)
    lo = (x - hi.astype(jnp.float32)).astype(jnp.bfloat16)
    dims = (((0,), (0,)), ((), ()))
    xt = (lax.dot_general(hi, eye, dims, preferred_element_type=jnp.float32)
          + lax.dot_general(lo, eye, dims,
                            preferred_element_type=jnp.float32))
    out_ref[:, 0:HIDDEN] = xt

  return pl.pallas_call(
      body,
      grid=(pl.cdiv(VOCAB, _TB),),
      in_specs=[pl.BlockSpec((HIDDEN, _TB), lambda i: (0, i))],
      out_specs=pl.BlockSpec((_TB, _HP), lambda i: (i, 0)),
      out_shape=jax.ShapeDtypeStruct((VOCAB, _HP), jnp.float32),
  )(emb_t)


def _make_gather():
  mesh = plsc.VectorSubcoreMesh(core_axis_name="c", subcore_axis_name="s")

  @functools.partial(
      pl.kernel,
      out_type=jax.ShapeDtypeStruct((BATCH, HIST, HIDDEN), jnp.float32),
      mesh=mesh,
      scratch_types=[
          pltpu.VMEM((_ISTAGE, HIST), jnp.int32),
          pltpu.VMEM((_ROWS_PER_W * HIST,), jnp.int32),
          pltpu.VMEM((_CHUNK, _HP), jnp.float32),
          pltpu.VMEM((_CHUNK, _HP), jnp.float32),
          pltpu.VMEM((_CHUNK, HIDDEN), jnp.float32),
          pltpu.VMEM((_CHUNK, HIDDEN), jnp.float32),
          pltpu.SemaphoreType.DMA,
          pltpu.SemaphoreType.DMA,
          pltpu.SemaphoreType.DMA,
          pltpu.SemaphoreType.DMA,
      ],
      compiler_params=pltpu.CompilerParams(
          use_tc_tiling_on_sc=True, disable_bounds_checks=True),
  )
  def gather_kernel(emb_hbm, idx_hbm, out_hbm, idx2d, idx_v,
                    rows0, rows1, sb0, sb1,
                    gsem0, gsem1, ssem0, ssem1):
    wid = lax.axis_index("s") * 2 + lax.axis_index("c")
    base = wid * _ROWS_PER_W
    fbase = wid * _B_PER_W
    out_flat = out_hbm.reshape(BATCH * HIST, HIDDEN)

    def stage_block(blk, _):
      pltpu.sync_copy(
          idx_hbm.at[pl.ds(base + blk * _ISTAGE, _ISTAGE), :], idx2d)

      def repack_row(r, _):
        b = blk * _ISTAGE + r
        for s in _CHS:
          idx_v[pl.ds(b * HIST + s, 16)] = idx2d[r, pl.ds(s, 16)]
        return 0

      lax.fori_loop(0, _ISTAGE, repack_row, 0, unroll=False)
      return 0

    lax.fori_loop(0, _ROWS_PER_W // _ISTAGE, stage_block, 0, unroll=False)

    rows = (rows0, rows1)
    sb = (sb0, sb1)
    gsem = (gsem0, gsem1)
    ssem = (ssem0, ssem1)

    def gather(i, b):
      pltpu.make_async_copy(
          emb_hbm.at[idx_v.at[pl.ds(i * _CHUNK, _CHUNK)]],
          rows[b], gsem[b]).start()

    def gather_wait(b):
      pltpu.make_async_copy(
          emb_hbm.at[idx_v.at[pl.ds(0, _CHUNK)]], rows[b], gsem[b]).wait()

    def repack(b):
      def row(r, _):
        for c in range(HIDDEN // 16):
          sb[b][r, pl.ds(c * 16, 16)] = rows[b][r, pl.ds(c * 16, 16)]
        return 0
      lax.fori_loop(0, _CHUNK, row, 0, unroll=False)

    def store(i, b):
      pltpu.make_async_copy(
          sb[b], out_flat.at[pl.ds(fbase + i * _CHUNK, _CHUNK)],
          ssem[b]).start()

    def store_wait(b):
      pltpu.make_async_copy(
          sb[b], out_flat.at[pl.ds(fbase, _CHUNK)], ssem[b]).wait()

    gather(0, 0)
    gather(1, 1)

    def body(i, b):
      gather_wait(b)

      @pl.when(i >= 2)
      def _():
        store_wait(b)

      repack(b)
      store(i, b)

      @pl.when(i + 2 < _NCHUNK)
      def _():
        gather(i + 2, b)

    def pair(k, _):
      g = 2 * k
      for b in range(2):
        body(g + b, b)
      return 0

    lax.fori_loop(0, _NCHUNK // 2, pair, 0, unroll=False)

    store_wait(0)
    store_wait(1)

  return gather_kernel


_gather = _make_gather()


def kernel(input_ids, emb):
  emb_pad = _transpose_pad(emb.T)
  return _gather(emb_pad, input_ids.astype(jnp.int32))


# speedup vs baseline: 2.1852x; 1.1455x over previous
import functools

import jax
import jax.numpy as jnp
from jax import lax
from jax.experimental import pallas as pl
from jax.experimental.pallas import tpu as pltpu
from jax.experimental.pallas import tpu_sc as plsc

VOCAB = 1_000_000
HIDDEN = 64
BATCH = 4096
HIST = 200

_NW = 32
_ROWS_PER_W = BATCH // _NW
_HP = 128
_CHUNK = 160
_B_PER_W = _ROWS_PER_W * HIST
_NCHUNK = _B_PER_W // _CHUNK
_TB = 4096
_ISTAGE = 32
_NCH = 13
_CHS = tuple(list(range(0, 192, 16)) + [184])


def _transpose_pad(emb_t):
  def body(in_ref, out_ref):
    x = in_ref[...]
    eye = jnp.eye(HIDDEN, dtype=jnp.bfloat16)
    hi = x.astype(jnp.bfloat16)
    lo = (x - hi.astype(jnp.float32)).astype(jnp.bfloat16)
    dims = (((0,), (0,)), ((), ()))
    xt = (lax.dot_general(hi, eye, dims, preferred_element_type=jnp.float32)
          + lax.dot_general(lo, eye, dims,
                            preferred_element_type=jnp.float32))
    out_ref[:, 0:HIDDEN] = xt

  return pl.pallas_call(
      body,
      grid=(pl.cdiv(VOCAB, _TB),),
      in_specs=[pl.BlockSpec((HIDDEN, _TB), lambda i: (0, i))],
      out_specs=pl.BlockSpec((_TB, _HP), lambda i: (i, 0)),
      out_shape=jax.ShapeDtypeStruct((VOCAB, _HP), jnp.float32),
  )(emb_t)


def _make_gather():
  mesh = plsc.VectorSubcoreMesh(core_axis_name="c", subcore_axis_name="s")

  @functools.partial(
      pl.kernel,
      out_type=jax.ShapeDtypeStruct((BATCH, HIST, HIDDEN), jnp.float32),
      mesh=mesh,
      scratch_types=[
          pltpu.VMEM((_ISTAGE, HIST), jnp.int32),
          pltpu.VMEM((_ROWS_PER_W * HIST,), jnp.int32),
          pltpu.VMEM((_CHUNK, _HP), jnp.float32),
          pltpu.VMEM((_CHUNK, _HP), jnp.float32),
          pltpu.VMEM((_CHUNK, HIDDEN), jnp.float32),
          pltpu.VMEM((_CHUNK, HIDDEN), jnp.float32),
          pltpu.SemaphoreType.DMA,
          pltpu.SemaphoreType.DMA,
          pltpu.SemaphoreType.DMA,
          pltpu.SemaphoreType.DMA,
      ],
      compiler_params=pltpu.CompilerParams(
          use_tc_tiling_on_sc=True, disable_bounds_checks=True),
  )
  def gather_kernel(emb_hbm, idx_hbm, out_hbm, idx2d, idx_v,
                    rows0, rows1, sb0, sb1,
                    gsem0, gsem1, ssem0, ssem1):
    wid = lax.axis_index("s") * 2 + lax.axis_index("c")
    base = wid * _ROWS_PER_W
    fbase = wid * _B_PER_W
    out_flat = out_hbm.reshape(BATCH * HIST, HIDDEN)

    def stage_block(blk, _):
      pltpu.sync_copy(
          idx_hbm.at[pl.ds(base + blk * _ISTAGE, _ISTAGE), :], idx2d)

      def repack_row(r, _):
        b = blk * _ISTAGE + r
        for s in _CHS:
          idx_v[pl.ds(b * HIST + s, 16)] = idx2d[r, pl.ds(s, 16)]
        return 0

      lax.fori_loop(0, _ISTAGE, repack_row, 0, unroll=False)
      return 0

    lax.fori_loop(0, _ROWS_PER_W // _ISTAGE, stage_block, 0, unroll=False)

    rows = (rows0, rows1)
    sb = (sb0, sb1)
    gsem = (gsem0, gsem1)
    ssem = (ssem0, ssem1)

    def gather(i, b):
      pltpu.make_async_copy(
          emb_hbm.at[idx_v.at[pl.ds(i * _CHUNK, _CHUNK)]],
          rows[b], gsem[b]).start()

    def gather_wait(b):
      pltpu.make_async_copy(
          emb_hbm.at[idx_v.at[pl.ds(0, _CHUNK)]], rows[b], gsem[b]).wait()

    def repack(b):
      def row(r, _):
        for rr in range(8):
          for c in range(HIDDEN // 16):
            sb[b][r * 8 + rr, pl.ds(c * 16, 16)] = (
                rows[b][r * 8 + rr, pl.ds(c * 16, 16)])
        return 0
      lax.fori_loop(0, _CHUNK // 8, row, 0, unroll=False)

    def store(i, b):
      pltpu.make_async_copy(
          sb[b], out_flat.at[pl.ds(fbase + i * _CHUNK, _CHUNK)],
          ssem[b]).start()

    def store_wait(b):
      pltpu.make_async_copy(
          sb[b], out_flat.at[pl.ds(fbase, _CHUNK)], ssem[b]).wait()

    gather(0, 0)
    gather(1, 1)

    def body(i, b):
      gather_wait(b)

      @pl.when(i >= 2)
      def _():
        store_wait(b)

      repack(b)
      store(i, b)

      @pl.when(i + 2 < _NCHUNK)
      def _():
        gather(i + 2, b)

    def pair(k, _):
      g = 2 * k
      for b in range(2):
        body(g + b, b)
      return 0

    lax.fori_loop(0, _NCHUNK // 2, pair, 0, unroll=False)

    store_wait(0)
    store_wait(1)

  return gather_kernel


_gather = _make_gather()


def kernel(input_ids, emb):
  emb_pad = _transpose_pad(emb.T)
  return _gather(emb_pad, input_ids.astype(jnp.int32))


# speedup vs baseline: 2.3799x; 1.0891x over previous
import functools

import jax
import jax.numpy as jnp
from jax import lax
from jax.experimental import pallas as pl
from jax.experimental.pallas import tpu as pltpu
from jax.experimental.pallas import tpu_sc as plsc

VOCAB = 1_000_000
HIDDEN = 64
BATCH = 4096
HIST = 200

_NW = 32
_ROWS_PER_W = BATCH // _NW
_HP = 128
_CHUNK = 160
_B_PER_W = _ROWS_PER_W * HIST
_NCHUNK = _B_PER_W // _CHUNK
_TB = 8192
_ISTAGE = 32
_NCH = 13
_CHS = tuple(list(range(0, 192, 16)) + [184])


def _transpose_pad(emb_t):
  def body(in_ref, out_ref):
    x = in_ref[...]
    eye = jnp.eye(HIDDEN, dtype=jnp.bfloat16)
    hi = x.astype(jnp.bfloat16)
    lo = (x - hi.astype(jnp.float32)).astype(jnp.bfloat16)
    dims = (((0,), (0,)), ((), ()))
    xt = (lax.dot_general(hi, eye, dims, preferred_element_type=jnp.float32)
          + lax.dot_general(lo, eye, dims,
                            preferred_element_type=jnp.float32))
    out_ref[:, 0:HIDDEN] = xt

  return pl.pallas_call(
      body,
      grid=(pl.cdiv(VOCAB, _TB),),
      in_specs=[pl.BlockSpec((HIDDEN, _TB), lambda i: (0, i))],
      out_specs=pl.BlockSpec((_TB, _HP), lambda i: (i, 0)),
      out_shape=jax.ShapeDtypeStruct((VOCAB, _HP), jnp.float32),
  )(emb_t)


def _make_gather():
  mesh = plsc.VectorSubcoreMesh(core_axis_name="c", subcore_axis_name="s")

  @functools.partial(
      pl.kernel,
      out_type=jax.ShapeDtypeStruct((BATCH, HIST, HIDDEN), jnp.float32),
      mesh=mesh,
      scratch_types=[
          pltpu.VMEM((_ISTAGE, HIST), jnp.int32),
          pltpu.VMEM((_ROWS_PER_W * HIST,), jnp.int32),
          pltpu.VMEM((_CHUNK, _HP), jnp.float32),
          pltpu.VMEM((_CHUNK, _HP), jnp.float32),
          pltpu.VMEM((_CHUNK, HIDDEN), jnp.float32),
          pltpu.VMEM((_CHUNK, HIDDEN), jnp.float32),
          pltpu.SemaphoreType.DMA,
          pltpu.SemaphoreType.DMA,
          pltpu.SemaphoreType.DMA,
          pltpu.SemaphoreType.DMA,
      ],
      compiler_params=pltpu.CompilerParams(
          use_tc_tiling_on_sc=True, disable_bounds_checks=True),
  )
  def gather_kernel(emb_hbm, idx_hbm, out_hbm, idx2d, idx_v,
                    rows0, rows1, sb0, sb1,
                    gsem0, gsem1, ssem0, ssem1):
    wid = lax.axis_index("s") * 2 + lax.axis_index("c")
    base = wid * _ROWS_PER_W
    fbase = wid * _B_PER_W
    out_flat = out_hbm.reshape(BATCH * HIST, HIDDEN)

    def stage_block(blk, _):
      pltpu.sync_copy(
          idx_hbm.at[pl.ds(base + blk * _ISTAGE, _ISTAGE), :], idx2d)

      def repack_row(r, _):
        b = blk * _ISTAGE + r
        for s in _CHS:
          idx_v[pl.ds(b * HIST + s, 16)] = idx2d[r, pl.ds(s, 16)]
        return 0

      lax.fori_loop(0, _ISTAGE, repack_row, 0, unroll=False)
      return 0

    lax.fori_loop(0, _ROWS_PER_W // _ISTAGE, stage_block, 0, unroll=False)

    rows = (rows0, rows1)
    sb = (sb0, sb1)
    gsem = (gsem0, gsem1)
    ssem = (ssem0, ssem1)

    def gather(i, b):
      pltpu.make_async_copy(
          emb_hbm.at[idx_v.at[pl.ds(i * _CHUNK, _CHUNK)]],
          rows[b], gsem[b]).start()

    def gather_wait(b):
      pltpu.make_async_copy(
          emb_hbm.at[idx_v.at[pl.ds(0, _CHUNK)]], rows[b], gsem[b]).wait()

    def repack(b):
      def row(r, _):
        for rr in range(8):
          for c in range(HIDDEN // 16):
            sb[b][r * 8 + rr, pl.ds(c * 16, 16)] = (
                rows[b][r * 8 + rr, pl.ds(c * 16, 16)])
        return 0
      lax.fori_loop(0, _CHUNK // 8, row, 0, unroll=False)

    def store(i, b):
      pltpu.make_async_copy(
          sb[b], out_flat.at[pl.ds(fbase + i * _CHUNK, _CHUNK)],
          ssem[b]).start()

    def store_wait(b):
      pltpu.make_async_copy(
          sb[b], out_flat.at[pl.ds(fbase, _CHUNK)], ssem[b]).wait()

    gather(0, 0)
    gather(1, 1)

    def body(i, b):
      gather_wait(b)

      @pl.when(i >= 2)
      def _():
        store_wait(b)

      repack(b)
      store(i, b)

      @pl.when(i + 2 < _NCHUNK)
      def _():
        gather(i + 2, b)

    def pair(k, _):
      g = 2 * k
      for b in range(2):
        body(g + b, b)
      return 0

    lax.fori_loop(0, _NCHUNK // 2, pair, 0, unroll=False)

    store_wait(0)
    store_wait(1)

  return gather_kernel


_gather = _make_gather()


def kernel(input_ids, emb):
  emb_pad = _transpose_pad(emb.T)
  return _gather(emb_pad, input_ids.astype(jnp.int32))


# speedup vs baseline: 2.4610x; 1.0341x over previous
import functools

import jax
import jax.numpy as jnp
from jax import lax
from jax.experimental import pallas as pl
from jax.experimental.pallas import tpu as pltpu
from jax.experimental.pallas import tpu_sc as plsc

VOCAB = 1_000_000
HIDDEN = 64
BATCH = 4096
HIST = 200

_NW = 32
_ROWS_PER_W = BATCH // _NW
_HP = 128
_CHUNK = 160
_B_PER_W = _ROWS_PER_W * HIST
_NCHUNK = _B_PER_W // _CHUNK
_TB = 16384
_ISTAGE = 32
_NCH = 13
_CHS = tuple(list(range(0, 192, 16)) + [184])


def _transpose_pad(emb_t):
  def body(in_ref, out_ref):
    x = in_ref[...]
    eye = jnp.eye(HIDDEN, dtype=jnp.bfloat16)
    hi = x.astype(jnp.bfloat16)
    lo = (x - hi.astype(jnp.float32)).astype(jnp.bfloat16)
    dims = (((0,), (0,)), ((), ()))
    xt = (lax.dot_general(hi, eye, dims, preferred_element_type=jnp.float32)
          + lax.dot_general(lo, eye, dims,
                            preferred_element_type=jnp.float32))
    out_ref[:, 0:HIDDEN] = xt

  return pl.pallas_call(
      body,
      grid=(pl.cdiv(VOCAB, _TB),),
      in_specs=[pl.BlockSpec((HIDDEN, _TB), lambda i: (0, i))],
      out_specs=pl.BlockSpec((_TB, _HP), lambda i: (i, 0)),
      out_shape=jax.ShapeDtypeStruct((VOCAB, _HP), jnp.float32),
  )(emb_t)


def _make_gather():
  mesh = plsc.VectorSubcoreMesh(core_axis_name="c", subcore_axis_name="s")

  @functools.partial(
      pl.kernel,
      out_type=jax.ShapeDtypeStruct((BATCH, HIST, HIDDEN), jnp.float32),
      mesh=mesh,
      scratch_types=[
          pltpu.VMEM((_ISTAGE, HIST), jnp.int32),
          pltpu.VMEM((_ROWS_PER_W * HIST,), jnp.int32),
          pltpu.VMEM((_CHUNK, _HP), jnp.float32),
          pltpu.VMEM((_CHUNK, _HP), jnp.float32),
          pltpu.VMEM((_CHUNK, HIDDEN), jnp.float32),
          pltpu.VMEM((_CHUNK, HIDDEN), jnp.float32),
          pltpu.SemaphoreType.DMA,
          pltpu.SemaphoreType.DMA,
          pltpu.SemaphoreType.DMA,
          pltpu.SemaphoreType.DMA,
      ],
      compiler_params=pltpu.CompilerParams(
          use_tc_tiling_on_sc=True, disable_bounds_checks=True),
  )
  def gather_kernel(emb_hbm, idx_hbm, out_hbm, idx2d, idx_v,
                    rows0, rows1, sb0, sb1,
                    gsem0, gsem1, ssem0, ssem1):
    wid = lax.axis_index("s") * 2 + lax.axis_index("c")
    base = wid * _ROWS_PER_W
    fbase = wid * _B_PER_W
    out_flat = out_hbm.reshape(BATCH * HIST, HIDDEN)

    def stage_block(blk, _):
      pltpu.sync_copy(
          idx_hbm.at[pl.ds(base + blk * _ISTAGE, _ISTAGE), :], idx2d)

      def repack_row(r, _):
        b = blk * _ISTAGE + r
        for s in _CHS:
          idx_v[pl.ds(b * HIST + s, 16)] = idx2d[r, pl.ds(s, 16)]
        return 0

      lax.fori_loop(0, _ISTAGE, repack_row, 0, unroll=False)
      return 0

    lax.fori_loop(0, _ROWS_PER_W // _ISTAGE, stage_block, 0, unroll=False)

    rows = (rows0, rows1)
    sb = (sb0, sb1)
    gsem = (gsem0, gsem1)
    ssem = (ssem0, ssem1)

    def gather(i, b):
      pltpu.make_async_copy(
          emb_hbm.at[idx_v.at[pl.ds(i * _CHUNK, _CHUNK)]],
          rows[b], gsem[b]).start()

    def gather_wait(b):
      pltpu.make_async_copy(
          emb_hbm.at[idx_v.at[pl.ds(0, _CHUNK)]], rows[b], gsem[b]).wait()

    def repack(b):
      def row(r, _):
        for rr in range(8):
          for c in range(HIDDEN // 16):
            sb[b][r * 8 + rr, pl.ds(c * 16, 16)] = (
                rows[b][r * 8 + rr, pl.ds(c * 16, 16)])
        return 0
      lax.fori_loop(0, _CHUNK // 8, row, 0, unroll=False)

    def store(i, b):
      pltpu.make_async_copy(
          sb[b], out_flat.at[pl.ds(fbase + i * _CHUNK, _CHUNK)],
          ssem[b]).start()

    def store_wait(b):
      pltpu.make_async_copy(
          sb[b], out_flat.at[pl.ds(fbase, _CHUNK)], ssem[b]).wait()

    gather(0, 0)
    gather(1, 1)

    def body(i, b):
      gather_wait(b)

      @pl.when(i >= 2)
      def _():
        store_wait(b)

      repack(b)
      store(i, b)

      @pl.when(i + 2 < _NCHUNK)
      def _():
        gather(i + 2, b)

    def pair(k, _):
      g = 2 * k
      for b in range(2):
        body(g + b, b)
      return 0

    lax.fori_loop(0, _NCHUNK // 2, pair, 0, unroll=False)

    store_wait(0)
    store_wait(1)

  return gather_kernel


_gather = _make_gather()


def kernel(input_ids, emb):
  emb_pad = _transpose_pad(emb.T)
  return _gather(emb_pad, input_ids.astype(jnp.int32))


# speedup vs baseline: 2.4779x; 1.0069x over previous
import functools

import jax
import jax.numpy as jnp
from jax import lax
from jax.experimental import pallas as pl
from jax.experimental.pallas import tpu as pltpu
from jax.experimental.pallas import tpu_sc as plsc

VOCAB = 1_000_000
HIDDEN = 64
BATCH = 4096
HIST = 200

_NW = 32
_ROWS_PER_W = BATCH // _NW
_HP = 128
_CHUNK = 160
_B_PER_W = _ROWS_PER_W * HIST
_NCHUNK = _B_PER_W // _CHUNK
_TB = 32768
_ISTAGE = 32
_NCH = 13
_CHS = tuple(list(range(0, 192, 16)) + [184])


def _transpose_pad(emb_t):
  def body(in_ref, out_ref):
    x = in_ref[...]
    eye = jnp.eye(HIDDEN, dtype=jnp.bfloat16)
    hi = x.astype(jnp.bfloat16)
    lo = (x - hi.astype(jnp.float32)).astype(jnp.bfloat16)
    dims = (((0,), (0,)), ((), ()))
    xt = (lax.dot_general(hi, eye, dims, preferred_element_type=jnp.float32)
          + lax.dot_general(lo, eye, dims,
                            preferred_element_type=jnp.float32))
    out_ref[:, 0:HIDDEN] = xt

  return pl.pallas_call(
      body,
      grid=(pl.cdiv(VOCAB, _TB),),
      in_specs=[pl.BlockSpec((HIDDEN, _TB), lambda i: (0, i))],
      out_specs=pl.BlockSpec((_TB, _HP), lambda i: (i, 0)),
      out_shape=jax.ShapeDtypeStruct((VOCAB, _HP), jnp.float32),
  )(emb_t)


def _make_gather():
  mesh = plsc.VectorSubcoreMesh(core_axis_name="c", subcore_axis_name="s")

  @functools.partial(
      pl.kernel,
      out_type=jax.ShapeDtypeStruct((BATCH, HIST, HIDDEN), jnp.float32),
      mesh=mesh,
      scratch_types=[
          pltpu.VMEM((_ISTAGE, HIST), jnp.int32),
          pltpu.VMEM((_ROWS_PER_W * HIST,), jnp.int32),
          pltpu.VMEM((_CHUNK, _HP), jnp.float32),
          pltpu.VMEM((_CHUNK, _HP), jnp.float32),
          pltpu.VMEM((_CHUNK, HIDDEN), jnp.float32),
          pltpu.VMEM((_CHUNK, HIDDEN), jnp.float32),
          pltpu.SemaphoreType.DMA,
          pltpu.SemaphoreType.DMA,
          pltpu.SemaphoreType.DMA,
          pltpu.SemaphoreType.DMA,
      ],
      compiler_params=pltpu.CompilerParams(
          use_tc_tiling_on_sc=True, disable_bounds_checks=True),
  )
  def gather_kernel(emb_hbm, idx_hbm, out_hbm, idx2d, idx_v,
                    rows0, rows1, sb0, sb1,
                    gsem0, gsem1, ssem0, ssem1):
    wid = lax.axis_index("s") * 2 + lax.axis_index("c")
    base = wid * _ROWS_PER_W
    fbase = wid * _B_PER_W
    out_flat = out_hbm.reshape(BATCH * HIST, HIDDEN)

    def stage_block(blk, _):
      pltpu.sync_copy(
          idx_hbm.at[pl.ds(base + blk * _ISTAGE, _ISTAGE), :], idx2d)

      def repack_row(r, _):
        b = blk * _ISTAGE + r
        for s in _CHS:
          idx_v[pl.ds(b * HIST + s, 16)] = idx2d[r, pl.ds(s, 16)]
        return 0

      lax.fori_loop(0, _ISTAGE, repack_row, 0, unroll=False)
      return 0

    lax.fori_loop(0, _ROWS_PER_W // _ISTAGE, stage_block, 0, unroll=False)

    rows = (rows0, rows1)
    sb = (sb0, sb1)
    gsem = (gsem0, gsem1)
    ssem = (ssem0, ssem1)

    def gather(i, b):
      pltpu.make_async_copy(
          emb_hbm.at[idx_v.at[pl.ds(i * _CHUNK, _CHUNK)]],
          rows[b], gsem[b]).start()

    def gather_wait(b):
      pltpu.make_async_copy(
          emb_hbm.at[idx_v.at[pl.ds(0, _CHUNK)]], rows[b], gsem[b]).wait()

    def repack(b):
      def row(r, _):
        for rr in range(8):
          for c in range(HIDDEN // 16):
            sb[b][r * 8 + rr, pl.ds(c * 16, 16)] = (
                rows[b][r * 8 + rr, pl.ds(c * 16, 16)])
        return 0
      lax.fori_loop(0, _CHUNK // 8, row, 0, unroll=False)

    def store(i, b):
      pltpu.make_async_copy(
          sb[b], out_flat.at[pl.ds(fbase + i * _CHUNK, _CHUNK)],
          ssem[b]).start()

    def store_wait(b):
      pltpu.make_async_copy(
          sb[b], out_flat.at[pl.ds(fbase, _CHUNK)], ssem[b]).wait()

    gather(0, 0)
    gather(1, 1)

    def body(i, b):
      gather_wait(b)

      @pl.when(i >= 2)
      def _():
        store_wait(b)

      repack(b)
      store(i, b)

      @pl.when(i + 2 < _NCHUNK)
      def _():
        gather(i + 2, b)

    def pair(k, _):
      g = 2 * k
      for b in range(2):
        body(g + b, b)
      return 0

    lax.fori_loop(0, _NCHUNK // 2, pair, 0, unroll=False)

    store_wait(0)
    store_wait(1)

  return gather_kernel


_gather = _make_gather()


def kernel(input_ids, emb):
  emb_pad = _transpose_pad(emb.T)
  return _gather(emb_pad, input_ids.astype(jnp.int32))
